# baseline (speedup 1.0000x reference)
import jax
import jax.numpy as jnp
from jax.experimental import pallas as pl
from jax.experimental.pallas import tpu as pltpu

BATCH = 10000
DIM = 512
CHUNKS = ((0, 1000), (1000, 8000), (9000, 1000))


def _stream_body(x_hbm, o_hbm, b0, b1, b2, sem_in, sem_out):
    bufs = (b0, b1, b2)

    def in_copy(j):
        s, n = CHUNKS[j]
        return pltpu.make_async_copy(
            x_hbm.at[pl.ds(s, n), :], bufs[j], sem_in.at[j]
        )

    def out_copy(j):
        s, n = CHUNKS[j]
        return pltpu.make_async_copy(
            bufs[j], o_hbm.at[pl.ds(s, n), :], sem_out.at[j]
        )

    in_copy(0).start()
    for j in range(3):
        in_copy(j).wait()
        out_copy(j).start()
        if j + 1 < 3:
            in_copy(j + 1).start()
    for j in range(3):
        out_copy(j).wait()


def kernel(x, ind, mask, sampled, embed):
    del ind, mask, sampled, embed
    return pl.pallas_call(
        _stream_body,
        in_specs=[pl.BlockSpec(memory_space=pltpu.MemorySpace.HBM)],
        out_specs=pl.BlockSpec(memory_space=pltpu.MemorySpace.HBM),
        out_shape=jax.ShapeDtypeStruct((BATCH, DIM), jnp.float32),
        scratch_shapes=[
            pltpu.VMEM((1000, DIM), jnp.float32),
            pltpu.VMEM((8000, DIM), jnp.float32),
            pltpu.VMEM((1000, DIM), jnp.float32),
            pltpu.SemaphoreType.DMA((3,)),
            pltpu.SemaphoreType.DMA((3,)),
        ],
    )(x)


# speedup vs baseline: 1.1972x; 1.1972x over previous
import jax
import jax.numpy as jnp
from jax.experimental import pallas as pl

BATCH = 10000
DIM = 512
ROWS_PER_BLOCK = 5000


def _copy_body(x_ref, o_ref):
    o_ref[...] = x_ref[...]


def kernel(x, ind, mask, sampled, embed):
    del ind, mask, sampled, embed
    return pl.pallas_call(
        _copy_body,
        grid=(BATCH // ROWS_PER_BLOCK,),
        in_specs=[pl.BlockSpec((ROWS_PER_BLOCK, DIM), lambda i: (i, 0))],
        out_specs=pl.BlockSpec((ROWS_PER_BLOCK, DIM), lambda i: (i, 0)),
        out_shape=jax.ShapeDtypeStruct((BATCH, DIM), jnp.float32),
    )(x)
